# Initial kernel scaffold; baseline (speedup 1.0000x reference)
import jax
import jax.numpy as jnp
from jax.experimental import pallas as pl


def kernel(x, ln1_g, ln1_b, ln2_g, ln2_b, Wq, Wk, Wv, Wp, bp, Wg, W1, b1, W2, b2):
    raise NotImplementedError("write your pallas kernel here")


# speedup vs baseline: 1.0228x; 1.0228x over previous
import functools

import jax
import jax.numpy as jnp
from jax import lax
from jax.experimental import pallas as pl

B, T, C, H, HD, E, K, F = 32, 256, 512, 8, 64, 2048 // 256, 2, 2048
E = 8
NEG = -1e30


def _attn_kernel(x_ref, wq_ref, wk_ref, wv_ref, wp_ref, bp_ref,
                 ln1g_ref, ln1b_ref, ln2g_ref, ln2b_ref,
                 x1_ref, h2_ref):
    x = x_ref[0]
    g1 = ln1g_ref[...]
    b1 = ln1b_ref[...]
    m = jnp.mean(x, axis=-1, keepdims=True)
    xc = x - m
    v = jnp.mean(xc * xc, axis=-1, keepdims=True)
    h = xc * lax.rsqrt(v + 1e-5) * g1 + b1

    q = jnp.dot(h, wq_ref[...], preferred_element_type=jnp.float32)
    k = jnp.dot(h, wk_ref[...], preferred_element_type=jnp.float32)
    vv = jnp.dot(h, wv_ref[...], preferred_element_type=jnp.float32)

    rows = lax.broadcasted_iota(jnp.int32, (T, T), 0)
    cols = lax.broadcasted_iota(jnp.int32, (T, T), 1)
    causal = rows >= cols
    scale = HD ** -0.5

    outs = []
    for hh in range(H):
        qh = q[:, hh * HD:(hh + 1) * HD]
        kh = k[:, hh * HD:(hh + 1) * HD]
        vh = vv[:, hh * HD:(hh + 1) * HD]
        s = lax.dot_general(qh, kh, (((1,), (1,)), ((), ())),
                            preferred_element_type=jnp.float32) * scale
        s = jnp.where(causal, s, NEG)
        mx = jnp.max(s, axis=-1, keepdims=True)
        ex = jnp.exp(s - mx)
        p = ex / jnp.sum(ex, axis=-1, keepdims=True)
        outs.append(jnp.dot(p, vh, preferred_element_type=jnp.float32))
    o = jnp.concatenate(outs, axis=-1)

    attn = jnp.dot(o, wp_ref[...], preferred_element_type=jnp.float32) + bp_ref[...]
    x1 = x + attn
    x1_ref[0] = x1

    m2 = jnp.mean(x1, axis=-1, keepdims=True)
    xc2 = x1 - m2
    v2 = jnp.mean(xc2 * xc2, axis=-1, keepdims=True)
    h2_ref[0] = xc2 * lax.rsqrt(v2 + 1e-5) * ln2g_ref[...] + ln2b_ref[...]


def _moe_dense_kernel(h2_ref, x1_ref, wg_ref, w1_ref, b1_ref, w2_ref, b2_ref,
                      out_ref):
    e = pl.program_id(1)
    h2 = h2_ref[...]
    TM = h2.shape[0]

    logits = jnp.dot(h2, wg_ref[...], preferred_element_type=jnp.float32)
    lane = lax.broadcasted_iota(jnp.int32, logits.shape, 1)
    logits = jnp.where(lane < E, logits, NEG)
    mx = jnp.max(logits, axis=-1, keepdims=True)
    ex = jnp.exp(logits - mx)
    w = ex / jnp.sum(ex, axis=-1, keepdims=True)
    m1 = jnp.max(w, axis=-1, keepdims=True)
    i1 = jnp.min(jnp.where(w == m1, lane, 128), axis=-1, keepdims=True)
    wmask = jnp.where(lane == i1, -1.0, w)
    m2 = jnp.max(wmask, axis=-1, keepdims=True)
    i2 = jnp.min(jnp.where(wmask == m2, lane, 128), axis=-1, keepdims=True)
    tot = m1 + m2
    g = jnp.where(i1 == e, m1 / tot, 0.0) + jnp.where(i2 == e, m2 / tot, 0.0)

    h1 = jnp.maximum(
        jnp.dot(h2, w1_ref[0], preferred_element_type=jnp.float32) + b1_ref[0], 0.0)
    oe = jnp.dot(h1, w2_ref[0], preferred_element_type=jnp.float32) + b2_ref[0]
    contrib = g * oe

    @pl.when(e == 0)
    def _():
        out_ref[...] = x1_ref[...] + contrib

    @pl.when(e != 0)
    def _():
        out_ref[...] = out_ref[...] + contrib


def kernel(x, ln1_g, ln1_b, ln2_g, ln2_b, Wq, Wk, Wv, Wp, bp, Wg, W1, b1, W2, b2):
    f32 = jnp.float32
    Wqr = Wq.transpose(1, 0, 2).reshape(C, H * HD)
    Wkr = Wk.transpose(1, 0, 2).reshape(C, H * HD)
    Wvr = Wv.transpose(1, 0, 2).reshape(C, H * HD)
    ln1g = ln1_g.reshape(1, C)
    ln1b = ln1_b.reshape(1, C)
    ln2g = ln2_g.reshape(1, C)
    ln2b = ln2_b.reshape(1, C)
    bpr = bp.reshape(1, C)

    x1, h2 = pl.pallas_call(
        _attn_kernel,
        grid=(B,),
        in_specs=[
            pl.BlockSpec((1, T, C), lambda b: (b, 0, 0)),
            pl.BlockSpec((C, H * HD), lambda b: (0, 0)),
            pl.BlockSpec((C, H * HD), lambda b: (0, 0)),
            pl.BlockSpec((C, H * HD), lambda b: (0, 0)),
            pl.BlockSpec((H * HD, C), lambda b: (0, 0)),
            pl.BlockSpec((1, C), lambda b: (0, 0)),
            pl.BlockSpec((1, C), lambda b: (0, 0)),
            pl.BlockSpec((1, C), lambda b: (0, 0)),
            pl.BlockSpec((1, C), lambda b: (0, 0)),
            pl.BlockSpec((1, C), lambda b: (0, 0)),
        ],
        out_specs=[
            pl.BlockSpec((1, T, C), lambda b: (b, 0, 0)),
            pl.BlockSpec((1, T, C), lambda b: (b, 0, 0)),
        ],
        out_shape=[
            jax.ShapeDtypeStruct((B, T, C), f32),
            jax.ShapeDtypeStruct((B, T, C), f32),
        ],
    )(x, Wqr, Wkr, Wvr, Wp, bpr, ln1g, ln1b, ln2g, ln2b)

    N = B * T
    TM = 256
    NT = N // TM
    h2f = h2.reshape(N, C)
    x1f = x1.reshape(N, C)
    Wg_pad = jnp.pad(Wg, ((0, 0), (0, 128 - E)))
    b1r = b1.reshape(E, 1, F)
    b2r = b2.reshape(E, 1, C)

    out = pl.pallas_call(
        _moe_dense_kernel,
        grid=(NT, E),
        in_specs=[
            pl.BlockSpec((TM, C), lambda t, e: (t, 0)),
            pl.BlockSpec((TM, C), lambda t, e: (t, 0)),
            pl.BlockSpec((C, 128), lambda t, e: (0, 0)),
            pl.BlockSpec((1, C, F), lambda t, e: (e, 0, 0)),
            pl.BlockSpec((1, 1, F), lambda t, e: (e, 0, 0)),
            pl.BlockSpec((1, F, C), lambda t, e: (e, 0, 0)),
            pl.BlockSpec((1, 1, C), lambda t, e: (e, 0, 0)),
        ],
        out_specs=pl.BlockSpec((TM, C), lambda t, e: (t, 0)),
        out_shape=jax.ShapeDtypeStruct((N, C), f32),
    )(h2f, x1f, Wg_pad, W1, b1r, W2, b2r)

    return out.reshape(B, T, C)


# speedup vs baseline: 1.4470x; 1.4148x over previous
import functools

import jax
import jax.numpy as jnp
from jax import lax
from jax.experimental import pallas as pl
from jax.experimental.pallas import tpu as pltpu
from jax.experimental.pallas import tpu_sc as plsc

B, T, C, H, HD, E, K, F = 32, 256, 512, 8, 64, 8, 2, 2048
N = B * T
NP = K * N
TM = 256
NB = N // TM
ROWS_PAD = 18432
NT_TILES = ROWS_PAD // TM
NEG = -1e30
NW = 32


def _attn_kernel(x_ref, wq_ref, wk_ref, wv_ref, wp_ref, bp_ref,
                 ln1g_ref, ln1b_ref, ln2g_ref, ln2b_ref,
                 x1_ref, h2_ref):
    x = x_ref[0]
    m = jnp.mean(x, axis=-1, keepdims=True)
    xc = x - m
    v = jnp.mean(xc * xc, axis=-1, keepdims=True)
    h = xc * lax.rsqrt(v + 1e-5) * ln1g_ref[...] + ln1b_ref[...]

    q = jnp.dot(h, wq_ref[...], preferred_element_type=jnp.float32)
    k = jnp.dot(h, wk_ref[...], preferred_element_type=jnp.float32)
    vv = jnp.dot(h, wv_ref[...], preferred_element_type=jnp.float32)

    rows = lax.broadcasted_iota(jnp.int32, (T, T), 0)
    cols = lax.broadcasted_iota(jnp.int32, (T, T), 1)
    causal = rows >= cols
    scale = HD ** -0.5

    outs = []
    for hh in range(H):
        qh = q[:, hh * HD:(hh + 1) * HD]
        kh = k[:, hh * HD:(hh + 1) * HD]
        vh = vv[:, hh * HD:(hh + 1) * HD]
        s = lax.dot_general(qh, kh, (((1,), (1,)), ((), ())),
                            preferred_element_type=jnp.float32) * scale
        s = jnp.where(causal, s, NEG)
        mx = jnp.max(s, axis=-1, keepdims=True)
        ex = jnp.exp(s - mx)
        p = ex / jnp.sum(ex, axis=-1, keepdims=True)
        outs.append(jnp.dot(p, vh, preferred_element_type=jnp.float32))
    o = jnp.concatenate(outs, axis=-1)

    attn = jnp.dot(o, wp_ref[...], preferred_element_type=jnp.float32) + bp_ref[...]
    x1 = x + attn
    x1_ref[0] = x1

    m2 = jnp.mean(x1, axis=-1, keepdims=True)
    xc2 = x1 - m2
    v2 = jnp.mean(xc2 * xc2, axis=-1, keepdims=True)
    h2_ref[0] = xc2 * lax.rsqrt(v2 + 1e-5) * ln2g_ref[...] + ln2b_ref[...]


def _router_kernel(h2_ref, wg_ref,
                   i1_ref, i2_ref, w0_ref, w1_ref, r0_ref, r1_ref, bs_ref):
    h2 = h2_ref[...]
    logits = jnp.dot(h2, wg_ref[...], preferred_element_type=jnp.float32)
    lane = lax.broadcasted_iota(jnp.int32, logits.shape, 1)
    logits = jnp.where(lane < E, logits, NEG)
    mx = jnp.max(logits, axis=-1, keepdims=True)
    ex = jnp.exp(logits - mx)
    w = ex / jnp.sum(ex, axis=-1, keepdims=True)
    m1 = jnp.max(w, axis=-1, keepdims=True)
    i1 = jnp.min(jnp.where(w == m1, lane, 128), axis=-1, keepdims=True)
    wmask = jnp.where(lane == i1, -1.0, w)
    m2 = jnp.max(wmask, axis=-1, keepdims=True)
    i2 = jnp.min(jnp.where(wmask == m2, lane, 128), axis=-1, keepdims=True)
    tot = m1 + m2

    p0 = (lane == i1).astype(jnp.float32)
    p1 = (lane == i2).astype(jnp.float32)

    ri = lax.broadcasted_iota(jnp.int32, (TM, TM), 0)
    ci = lax.broadcasted_iota(jnp.int32, (TM, TM), 1)
    tris = (ci < ri).astype(jnp.float32)

    r0 = lax.dot_general(tris, p0, (((1,), (0,)), ((), ())),
                         preferred_element_type=jnp.float32)
    bsum0 = jnp.sum(p0, axis=0, keepdims=True)
    r1 = lax.dot_general(tris, p1, (((1,), (0,)), ((), ())),
                         preferred_element_type=jnp.float32) + bsum0

    i1_ref[...] = i1
    i2_ref[...] = i2
    w0_ref[...] = m1 / tot
    w1_ref[...] = m2 / tot
    r0_ref[...] = jnp.sum(p0 * r0, axis=-1, keepdims=True)
    r1_ref[...] = jnp.sum(p1 * r1, axis=-1, keepdims=True)
    bs_ref[0] = bsum0 + jnp.sum(p1, axis=0, keepdims=True)


def _offsets_kernel(bs_ref, bo_ref, off_ref, te_ref):
    bs = bs_ref[...].reshape(NB, 128)
    ri = lax.broadcasted_iota(jnp.int32, (NB, NB), 0)
    ci = lax.broadcasted_iota(jnp.int32, (NB, NB), 1)
    tris = (ci < ri).astype(jnp.float32)
    blockoff = lax.dot_general(tris, bs, (((1,), (0,)), ((), ())),
                               preferred_element_type=jnp.float32)
    counts = jnp.sum(bs, axis=0, keepdims=True)
    aligned = jnp.floor((counts + (TM - 1.0)) / TM) * TM

    ri2 = lax.broadcasted_iota(jnp.int32, (128, 128), 0)
    ci2 = lax.broadcasted_iota(jnp.int32, (128, 128), 1)
    upper = (ri2 < ci2).astype(jnp.float32)
    off = jnp.dot(aligned, upper, preferred_element_type=jnp.float32)

    ident = (ri2 == ci2).astype(jnp.float32)
    off_col = lax.dot_general(ident, off, (((1,), (1,)), ((), ())),
                              preferred_element_type=jnp.float32)
    nt_col = off_col * (1.0 / TM)
    jrow = lax.broadcasted_iota(jnp.int32, (1, 128), 1).astype(jnp.float32)
    esel = ((ri2 >= 1) & (ri2 < E)).astype(jnp.float32)
    cmp = jnp.where(nt_col <= jrow, 1.0, 0.0) * esel
    te = jnp.dot(jnp.ones((1, 128), jnp.float32), cmp,
                 preferred_element_type=jnp.float32)

    bo_ref[...] = blockoff.reshape(NB, 1, 128)
    off_ref[...] = off
    te_ref[...] = te.astype(jnp.int32)


def _dest_kernel(i1_ref, i2_ref, r0_ref, r1_ref, bo_ref, off_ref,
                 d0_ref, d1_ref):
    lane = lax.broadcasted_iota(jnp.int32, (TM, 128), 1)
    off = off_ref[...]
    bo = bo_ref[0]
    p0 = (lane == i1_ref[...]).astype(jnp.float32)
    p1 = (lane == i2_ref[...]).astype(jnp.float32)
    d0 = jnp.sum(p0 * (off + bo), axis=-1, keepdims=True) + r0_ref[...]
    d1 = jnp.sum(p1 * (off + bo), axis=-1, keepdims=True) + r1_ref[...]
    d0_ref[...] = d0.astype(jnp.int32)
    d1_ref[...] = d1.astype(jnp.int32)


def _make_scatter_tokens():
    mesh = plsc.VectorSubcoreMesh(core_axis_name="c", subcore_axis_name="s", num_cores=2, num_subcores=16)

    @functools.partial(
        pl.kernel, mesh=mesh,
        out_type=jax.ShapeDtypeStruct((ROWS_PAD,), jnp.int32),
        scratch_types=[
            pltpu.VMEM((NP,), jnp.int32),
            pltpu.VMEM((ROWS_PAD,), jnp.int32),
        ],
        compiler_params=pltpu.CompilerParams(needs_layout_passes=False),
    )
    def scatter_k(dest_hbm, srcidx_hbm, d_v, si_v):
        cid = lax.axis_index("c")
        sid = lax.axis_index("s")

        @pl.when((cid == 0) & (sid == 0))
        def _():
            pltpu.sync_copy(dest_hbm, d_v)

            def zbody(i, carry):
                si_v[pl.ds(i * 16, 16)] = jnp.zeros((16,), jnp.int32)
                return carry

            lax.fori_loop(0, ROWS_PAD // 16, zbody, 0)

            def sbody(i, carry):
                idx = d_v[pl.ds(i * 16, 16)]
                p = i * 16 + lax.iota(jnp.int32, 16)
                tok = lax.bitwise_and(p, N - 1)
                plsc.store_scatter(si_v, [idx], tok)
                return carry

            lax.fori_loop(0, NP // 16, sbody, 0)
            pltpu.sync_copy(si_v, srcidx_hbm)

    return scatter_k


def _make_row_gather(n_rows, n_src_rows, chunk):
    del n_src_rows
    rows_per_w = n_rows // NW
    n_chunks = rows_per_w // chunk
    mesh = plsc.VectorSubcoreMesh(core_axis_name="c", subcore_axis_name="s", num_cores=2, num_subcores=16)

    @functools.partial(
        pl.kernel, mesh=mesh,
        out_type=jax.ShapeDtypeStruct((n_rows, C), jnp.float32),
        scratch_types=[
            pltpu.VMEM((rows_per_w,), jnp.int32),
            pltpu.VMEM((chunk, C), jnp.float32),
            pltpu.SemaphoreType.DMA,
        ],
        compiler_params=pltpu.CompilerParams(needs_layout_passes=False),
    )
    def gather_k(src_hbm, idx_hbm, out_hbm, idx_v, rows_v, sem):
        wid = lax.axis_index("s") * 2 + lax.axis_index("c")
        base = wid * rows_per_w
        pltpu.sync_copy(idx_hbm.at[pl.ds(base, rows_per_w)], idx_v)

        def body(j, carry):
            pltpu.async_copy(
                src_hbm.at[idx_v.at[pl.ds(j * chunk, chunk)]], rows_v, sem
            ).wait()
            pltpu.sync_copy(rows_v, out_hbm.at[pl.ds(base + j * chunk, chunk)])
            return carry

        lax.fori_loop(0, n_chunks, body, 0)

    return gather_k


def _expert_kernel(te_ref, xe_ref, w1_ref, b1_ref, w2_ref, b2_ref, out_ref):
    del te_ref
    xe = xe_ref[...]
    h1 = jnp.maximum(
        jnp.dot(xe, w1_ref[0], preferred_element_type=jnp.float32) + b1_ref[0],
        0.0)
    out_ref[...] = (
        jnp.dot(h1, w2_ref[0], preferred_element_type=jnp.float32) + b2_ref[0])


def _combine_kernel(x1_ref, y0_ref, y1_ref, w0_ref, w1_ref, out_ref):
    out_ref[...] = (x1_ref[...]
                    + w0_ref[...] * y0_ref[...]
                    + w1_ref[...] * y1_ref[...])


def kernel(x, ln1_g, ln1_b, ln2_g, ln2_b, Wq, Wk, Wv, Wp, bp, Wg, W1, b1, W2, b2):
    f32 = jnp.float32
    Wqr = Wq.transpose(1, 0, 2).reshape(C, H * HD)
    Wkr = Wk.transpose(1, 0, 2).reshape(C, H * HD)
    Wvr = Wv.transpose(1, 0, 2).reshape(C, H * HD)
    ln1g = ln1_g.reshape(1, C)
    ln1b = ln1_b.reshape(1, C)
    ln2g = ln2_g.reshape(1, C)
    ln2b = ln2_b.reshape(1, C)
    bpr = bp.reshape(1, C)

    x1, h2 = pl.pallas_call(
        _attn_kernel,
        grid=(B,),
        in_specs=[
            pl.BlockSpec((1, T, C), lambda b: (b, 0, 0)),
            pl.BlockSpec((C, H * HD), lambda b: (0, 0)),
            pl.BlockSpec((C, H * HD), lambda b: (0, 0)),
            pl.BlockSpec((C, H * HD), lambda b: (0, 0)),
            pl.BlockSpec((H * HD, C), lambda b: (0, 0)),
            pl.BlockSpec((1, C), lambda b: (0, 0)),
            pl.BlockSpec((1, C), lambda b: (0, 0)),
            pl.BlockSpec((1, C), lambda b: (0, 0)),
            pl.BlockSpec((1, C), lambda b: (0, 0)),
            pl.BlockSpec((1, C), lambda b: (0, 0)),
        ],
        out_specs=[
            pl.BlockSpec((1, T, C), lambda b: (b, 0, 0)),
            pl.BlockSpec((1, T, C), lambda b: (b, 0, 0)),
        ],
        out_shape=[
            jax.ShapeDtypeStruct((B, T, C), f32),
            jax.ShapeDtypeStruct((B, T, C), f32),
        ],
    )(x, Wqr, Wkr, Wvr, Wp, bpr, ln1g, ln1b, ln2g, ln2b)

    h2f = h2.reshape(N, C)
    x1f = x1.reshape(N, C)
    Wg_pad = jnp.pad(Wg, ((0, 0), (0, 128 - E)))

    i1o, i2o, w0o, w1o, r0o, r1o, bso = pl.pallas_call(
        _router_kernel,
        grid=(NB,),
        in_specs=[
            pl.BlockSpec((TM, C), lambda b: (b, 0)),
            pl.BlockSpec((C, 128), lambda b: (0, 0)),
        ],
        out_specs=[
            pl.BlockSpec((TM, 1), lambda b: (b, 0)),
            pl.BlockSpec((TM, 1), lambda b: (b, 0)),
            pl.BlockSpec((TM, 1), lambda b: (b, 0)),
            pl.BlockSpec((TM, 1), lambda b: (b, 0)),
            pl.BlockSpec((TM, 1), lambda b: (b, 0)),
            pl.BlockSpec((TM, 1), lambda b: (b, 0)),
            pl.BlockSpec((1, 1, 128), lambda b: (b, 0, 0)),
        ],
        out_shape=[
            jax.ShapeDtypeStruct((N, 1), jnp.int32),
            jax.ShapeDtypeStruct((N, 1), jnp.int32),
            jax.ShapeDtypeStruct((N, 1), f32),
            jax.ShapeDtypeStruct((N, 1), f32),
            jax.ShapeDtypeStruct((N, 1), f32),
            jax.ShapeDtypeStruct((N, 1), f32),
            jax.ShapeDtypeStruct((NB, 1, 128), f32),
        ],
    )(h2f, Wg_pad)

    boo, offo, teo = pl.pallas_call(
        _offsets_kernel,
        grid=(1,),
        in_specs=[pl.BlockSpec((NB, 1, 128), lambda i: (0, 0, 0))],
        out_specs=[
            pl.BlockSpec((NB, 1, 128), lambda i: (0, 0, 0)),
            pl.BlockSpec((1, 128), lambda i: (0, 0)),
            pl.BlockSpec((1, 128), lambda i: (0, 0)),
        ],
        out_shape=[
            jax.ShapeDtypeStruct((NB, 1, 128), f32),
            jax.ShapeDtypeStruct((1, 128), f32),
            jax.ShapeDtypeStruct((1, 128), jnp.int32),
        ],
    )(bso)

    d0o, d1o = pl.pallas_call(
        _dest_kernel,
        grid=(NB,),
        in_specs=[
            pl.BlockSpec((TM, 1), lambda b: (b, 0)),
            pl.BlockSpec((TM, 1), lambda b: (b, 0)),
            pl.BlockSpec((TM, 1), lambda b: (b, 0)),
            pl.BlockSpec((TM, 1), lambda b: (b, 0)),
            pl.BlockSpec((1, 1, 128), lambda b: (b, 0, 0)),
            pl.BlockSpec((1, 128), lambda b: (0, 0)),
        ],
        out_specs=[
            pl.BlockSpec((TM, 1), lambda b: (b, 0)),
            pl.BlockSpec((TM, 1), lambda b: (b, 0)),
        ],
        out_shape=[
            jax.ShapeDtypeStruct((N, 1), jnp.int32),
            jax.ShapeDtypeStruct((N, 1), jnp.int32),
        ],
    )(i1o, i2o, r0o, r1o, boo, offo)

    dest = jnp.concatenate([d0o, d1o], axis=0).reshape(NP)

    src_idx = _make_scatter_tokens()(dest)

    xe = _make_row_gather(ROWS_PAD, N, 96)(h2f, src_idx)

    b1r = b1.reshape(E, 1, F)
    b2r = b2.reshape(E, 1, C)
    tile_e = teo.reshape(128)

    grid_spec = pltpu.PrefetchScalarGridSpec(
        num_scalar_prefetch=1,
        grid=(NT_TILES,),
        in_specs=[
            pl.BlockSpec((TM, C), lambda j, s: (j, 0)),
            pl.BlockSpec((1, C, F), lambda j, s: (s[j], 0, 0)),
            pl.BlockSpec((1, 1, F), lambda j, s: (s[j], 0, 0)),
            pl.BlockSpec((1, F, C), lambda j, s: (s[j], 0, 0)),
            pl.BlockSpec((1, 1, C), lambda j, s: (s[j], 0, 0)),
        ],
        out_specs=pl.BlockSpec((TM, C), lambda j, s: (j, 0)),
    )
    ye = pl.pallas_call(
        _expert_kernel,
        grid_spec=grid_spec,
        out_shape=jax.ShapeDtypeStruct((ROWS_PAD, C), f32),
    )(tile_e, xe, W1, b1r, W2, b2r)

    yg = _make_row_gather(NP, ROWS_PAD, 128)(ye, dest)

    out = pl.pallas_call(
        _combine_kernel,
        grid=(NB,),
        in_specs=[
            pl.BlockSpec((TM, C), lambda b: (b, 0)),
            pl.BlockSpec((TM, C), lambda b: (b, 0)),
            pl.BlockSpec((TM, C), lambda b: (b + NB, 0)),
            pl.BlockSpec((TM, 1), lambda b: (b, 0)),
            pl.BlockSpec((TM, 1), lambda b: (b, 0)),
        ],
        out_specs=pl.BlockSpec((TM, C), lambda b: (b, 0)),
        out_shape=jax.ShapeDtypeStruct((N, C), f32),
    )(x1f, yg, yg, w0o, w1o)

    return out.reshape(B, T, C)
